# Initial kernel scaffold; baseline (speedup 1.0000x reference)
import jax
import jax.numpy as jnp
from jax.experimental import pallas as pl


def kernel(embeddings, targets, centers):
    raise NotImplementedError("write your pallas kernel here")


# speedup vs baseline: 4.3702x; 4.3702x over previous
import jax
import jax.numpy as jnp
from jax.experimental import pallas as pl
from jax.experimental.pallas import tpu as pltpu

LAMBD_ = 0.5
CPAD = 1024


def _loss_body(e_ref, t_ref, ct_ref, tot_ref, cnt_ref):
    i = pl.program_id(0)

    @pl.when(i == 0)
    def _init():
        tot_ref[...] = jnp.zeros_like(tot_ref)
        cnt_ref[...] = jnp.zeros_like(cnt_ref)

    e = e_ref[...]
    tgt = t_ref[...]
    ct = ct_ref[...]
    c2 = jnp.sum(ct * ct, axis=0, keepdims=True)

    e2 = jnp.sum(e * e, axis=1, keepdims=True)
    dot = jnp.dot(e, ct, preferred_element_type=jnp.float32)
    dist = (e2 + c2) - 2.0 * dot

    col = jax.lax.broadcasted_iota(jnp.int32, dist.shape, 1)
    onehot = col == tgt
    ap = jnp.sum(jnp.where(onehot, dist, 0.0), axis=1, keepdims=True)

    diff = (LAMBD_ + ap) - dist
    valid = (diff > 0.0) & (col < 1000) & jnp.logical_not(onehot)
    tot_ref[...] += jnp.sum(jnp.where(valid, diff, 0.0)).reshape(1, 1)
    cnt_ref[...] += jnp.sum(valid.astype(jnp.float32)).reshape(1, 1)


def kernel(embeddings, targets, centers):
    n, d = embeddings.shape
    c = centers.shape[0]
    bn = 512
    ct = jnp.pad(centers, ((0, CPAD - c), (0, 0))).T
    tgt = targets.astype(jnp.int32).reshape(n, 1)

    tot, cnt = pl.pallas_call(
        _loss_body,
        grid=(n // bn,),
        in_specs=[
            pl.BlockSpec((bn, d), lambda i: (i, 0)),
            pl.BlockSpec((bn, 1), lambda i: (i, 0)),
            pl.BlockSpec((d, CPAD), lambda i: (0, 0)),
        ],
        out_specs=[
            pl.BlockSpec((1, 1), lambda i: (0, 0)),
            pl.BlockSpec((1, 1), lambda i: (0, 0)),
        ],
        out_shape=[
            jax.ShapeDtypeStruct((1, 1), jnp.float32),
            jax.ShapeDtypeStruct((1, 1), jnp.float32),
        ],
    )(embeddings, tgt, ct)

    total = tot[0, 0]
    count = cnt[0, 0]
    denom = jnp.maximum(count, 1.0)
    return jnp.where(count > 0, total / denom, jnp.zeros((), jnp.float32))


# speedup vs baseline: 4.8002x; 1.0984x over previous
import jax
import jax.numpy as jnp
from jax.experimental import pallas as pl
from jax.experimental.pallas import tpu as pltpu

LAMBD_ = 0.5
CPAD = 1024
SENTINEL = 1.0e5


def _loss_body(e_ref, t_ref, ct_ref, col_ref, tot_ref, cnt_ref):
    i = pl.program_id(0)

    @pl.when(i == 0)
    def _init():
        tot_ref[...] = jnp.zeros_like(tot_ref)
        cnt_ref[...] = jnp.zeros_like(cnt_ref)

    e = e_ref[...]
    tgt = t_ref[...]
    ct = ct_ref[...]
    col = col_ref[...]

    c2 = jnp.sum(ct * ct, axis=0, keepdims=True)
    dot = jnp.dot(e, ct, preferred_element_type=jnp.float32)
    u = c2 - 2.0 * dot

    onehot = col == tgt
    uat = jnp.sum(jnp.where(onehot, u, 0.0), axis=1, keepdims=True)
    diff = (LAMBD_ + uat) - u
    tot_ref[...] += jnp.sum(jnp.maximum(diff, 0.0)).reshape(1, 1)
    cnt_ref[...] += jnp.sum((diff > 0.0).astype(jnp.float32)).reshape(1, 1)


def kernel(embeddings, targets, centers):
    n, d = embeddings.shape
    c = centers.shape[0]
    bn = 512
    ct = jnp.pad(centers, ((0, CPAD - c), (0, 0)),
                 constant_values=SENTINEL).T
    tgt = targets.astype(jnp.int32).reshape(n, 1)
    col = jax.lax.iota(jnp.int32, CPAD).reshape(1, CPAD)

    tot, cnt = pl.pallas_call(
        _loss_body,
        grid=(n // bn,),
        in_specs=[
            pl.BlockSpec((bn, d), lambda i: (i, 0)),
            pl.BlockSpec((bn, 1), lambda i: (i, 0)),
            pl.BlockSpec((d, CPAD), lambda i: (0, 0)),
            pl.BlockSpec((1, CPAD), lambda i: (0, 0)),
        ],
        out_specs=[
            pl.BlockSpec((1, 1), lambda i: (0, 0)),
            pl.BlockSpec((1, 1), lambda i: (0, 0)),
        ],
        out_shape=[
            jax.ShapeDtypeStruct((1, 1), jnp.float32),
            jax.ShapeDtypeStruct((1, 1), jnp.float32),
        ],
    )(embeddings, tgt, ct, col)

    total = tot[0, 0] - n * LAMBD_
    count = cnt[0, 0] - n
    denom = jnp.maximum(count, 1.0)
    return jnp.where(count > 0, total / denom, jnp.zeros((), jnp.float32))


# speedup vs baseline: 5.0419x; 1.0503x over previous
import jax
import jax.numpy as jnp
from jax.experimental import pallas as pl
from jax.experimental.pallas import tpu as pltpu

LAMBD_ = 0.5
CPAD = 1024
SENTINEL = 1.0e5


def _loss_body(e_ref, t_ref, ct_ref, col_ref, tot_ref, cnt_ref):
    i = pl.program_id(0)

    @pl.when(i == 0)
    def _init():
        tot_ref[...] = jnp.zeros_like(tot_ref)
        cnt_ref[...] = jnp.zeros_like(cnt_ref)

    e = e_ref[...]
    tgt = t_ref[...]
    ct = ct_ref[...]
    col = col_ref[...]

    c2 = jnp.sum(ct * ct, axis=0, keepdims=True)
    dot = jnp.dot(e, ct, preferred_element_type=jnp.float32)
    u = c2 - 2.0 * dot

    onehot = col == tgt
    uat = jnp.sum(jnp.where(onehot, u, 0.0), axis=1, keepdims=True)
    diff = (LAMBD_ + uat) - u
    tot_ref[...] += jnp.sum(jnp.maximum(diff, 0.0)).reshape(1, 1)
    cnt_ref[...] += jnp.sum((diff > 0.0).astype(jnp.float32)).reshape(1, 1)


def kernel(embeddings, targets, centers):
    n, d = embeddings.shape
    c = centers.shape[0]
    bn = 4096
    ct = jnp.pad(centers, ((0, CPAD - c), (0, 0)),
                 constant_values=SENTINEL).T
    tgt = targets.astype(jnp.int32).reshape(n, 1)
    col = jax.lax.iota(jnp.int32, CPAD).reshape(1, CPAD)

    tot, cnt = pl.pallas_call(
        _loss_body,
        grid=(n // bn,),
        in_specs=[
            pl.BlockSpec((bn, d), lambda i: (i, 0)),
            pl.BlockSpec((bn, 1), lambda i: (i, 0)),
            pl.BlockSpec((d, CPAD), lambda i: (0, 0)),
            pl.BlockSpec((1, CPAD), lambda i: (0, 0)),
        ],
        out_specs=[
            pl.BlockSpec((1, 1), lambda i: (0, 0)),
            pl.BlockSpec((1, 1), lambda i: (0, 0)),
        ],
        out_shape=[
            jax.ShapeDtypeStruct((1, 1), jnp.float32),
            jax.ShapeDtypeStruct((1, 1), jnp.float32),
        ],
    )(embeddings, tgt, ct, col)

    total = tot[0, 0] - n * LAMBD_
    count = cnt[0, 0] - n
    denom = jnp.maximum(count, 1.0)
    return jnp.where(count > 0, total / denom, jnp.zeros((), jnp.float32))


# speedup vs baseline: 6.2808x; 1.2457x over previous
import jax
import jax.numpy as jnp
from jax.experimental import pallas as pl
from jax.experimental.pallas import tpu as pltpu

LAMBD_ = 0.5
CPAD = 1024
SENTINEL = 1.0e5


def _loss_body(e_ref, t_ref, c_ref, col_ref, out_ref, ct_s):
    cen = c_ref[...]
    n, c = e_ref.shape[0], cen.shape[0]

    ct_s[...] = jnp.full(ct_s.shape, SENTINEL, jnp.float32)
    ct_s[:, :c] = cen.T

    e = e_ref[...]
    tgt = t_ref[...]
    ct = ct_s[...]
    col = col_ref[...]

    c2 = jnp.sum(ct * ct, axis=0, keepdims=True)
    dot = jnp.dot(e, ct, preferred_element_type=jnp.float32)
    u = c2 - 2.0 * dot

    onehot = col == tgt
    uat = jnp.sum(jnp.where(onehot, u, 0.0), axis=1, keepdims=True)
    diff = (LAMBD_ + uat) - u
    total = jnp.sum(jnp.maximum(diff, 0.0)) - n * LAMBD_
    count = jnp.sum((diff > 0.0).astype(jnp.float32)) - n
    loss = jnp.where(count > 0, total / jnp.maximum(count, 1.0), 0.0)
    out_ref[...] = loss.reshape(1, 1)


def kernel(embeddings, targets, centers):
    n, d = embeddings.shape
    tgt = targets.astype(jnp.int32).reshape(n, 1)
    col = jax.lax.iota(jnp.int32, CPAD).reshape(1, CPAD)

    out = pl.pallas_call(
        _loss_body,
        out_shape=jax.ShapeDtypeStruct((1, 1), jnp.float32),
        scratch_shapes=[pltpu.VMEM((d, CPAD), jnp.float32)],
    )(embeddings, tgt, centers, col)

    return out[0, 0]


# speedup vs baseline: 6.3496x; 1.0110x over previous
import jax
import jax.numpy as jnp
from jax.experimental import pallas as pl
from jax.experimental.pallas import tpu as pltpu

LAMBD_ = 0.5
CPAD = 1024
SENTINEL = 1.0e5
BN = 1024


def _loss_body(e_ref, t_ref, c_ref, col_ref, out_ref, ct_s, c2_s, tot_s, cnt_s):
    i = pl.program_id(0)
    nsteps = pl.num_programs(0)
    c = c_ref.shape[0]

    @pl.when(i == 0)
    def _init():
        ct_s[...] = jnp.full(ct_s.shape, SENTINEL, jnp.float32)
        ct_s[:, :c] = c_ref[...].T
        ct0 = ct_s[...]
        c2_s[...] = jnp.sum(ct0 * ct0, axis=0, keepdims=True)
        tot_s[...] = jnp.zeros_like(tot_s)
        cnt_s[...] = jnp.zeros_like(cnt_s)

    e = e_ref[...]
    tgt = t_ref[...]
    ct = ct_s[...]
    col = col_ref[...]
    c2 = c2_s[...]

    dot = jnp.dot(e, ct, preferred_element_type=jnp.float32)
    u = c2 - 2.0 * dot

    onehot = col == tgt
    uat = jnp.sum(jnp.where(onehot, u, 0.0), axis=1, keepdims=True)
    diff = (LAMBD_ + uat) - u
    pos = diff > 0.0
    tot_s[...] += jnp.sum(jnp.where(pos, diff, 0.0)).reshape(1, 1)
    cnt_s[...] += jnp.sum(pos.astype(jnp.float32)).reshape(1, 1)

    @pl.when(i == nsteps - 1)
    def _fin():
        n = e_ref.shape[0] * nsteps
        total = tot_s[0, 0] - n * LAMBD_
        count = cnt_s[0, 0] - n
        loss = jnp.where(count > 0, total / jnp.maximum(count, 1.0), 0.0)
        out_ref[...] = loss.reshape(1, 1)


def kernel(embeddings, targets, centers):
    n, d = embeddings.shape
    c = centers.shape[0]
    tgt = targets.astype(jnp.int32).reshape(n, 1)
    col = jax.lax.iota(jnp.int32, CPAD).reshape(1, CPAD)

    out = pl.pallas_call(
        _loss_body,
        grid=(n // BN,),
        in_specs=[
            pl.BlockSpec((BN, d), lambda i: (i, 0)),
            pl.BlockSpec((BN, 1), lambda i: (i, 0)),
            pl.BlockSpec((c, d), lambda i: (0, 0)),
            pl.BlockSpec((1, CPAD), lambda i: (0, 0)),
        ],
        out_specs=pl.BlockSpec((1, 1), lambda i: (0, 0)),
        out_shape=jax.ShapeDtypeStruct((1, 1), jnp.float32),
        scratch_shapes=[
            pltpu.VMEM((d, CPAD), jnp.float32),
            pltpu.VMEM((1, CPAD), jnp.float32),
            pltpu.VMEM((1, 1), jnp.float32),
            pltpu.VMEM((1, 1), jnp.float32),
        ],
    )(embeddings, tgt, centers, col)

    return out[0, 0]
